# baseline (speedup 1.0000x reference)
import functools

import jax
import jax.numpy as jnp
from jax import lax
from jax.experimental import pallas as pl
from jax.experimental.pallas import tpu as pltpu
from jax.experimental.pallas import tpu_sc as plsc

_BATCH = 2048
_IN_DIM = 8192
_OUT_DIM = 8192
_TAU = 1.0

_NC = 2
_NS = 16
_NW = _NC * _NS
_L = 16

_ROWS_PER_W = _BATCH // _NW
_R = 3
_NCHUNK = _OUT_DIM // _L
_STEPS = [(3 * k, 3) for k in range(21)] + [(63, 1)]

_SLICE = _OUT_DIM // _NS
_NGRP = _SLICE // _L


def _round_bf16_lo(u):
    return lax.shift_right_logical(u + jnp.int32(0x8000), 16)


def _round_bf16_hi(u):
    return jnp.bitwise_and(u + jnp.int32(0x8000), jnp.int32(-65536))


def _sc_body(x_hbm, w_hbm, ia_hbm, ib_hbm, out_hbm,
             pk_v, pc1_v, pc2_v,
             iasl_v, ibsl_v, pksl_v, pc1sl_v, pc2sl_v,
             pk_sh, pc1_sh, pc2_sh,
             xin00, xin01, xin02, xin10, xin11, xin12,
             yo00, yo01, yo02, yo10, yo11, yo12,
             in_sem0, in_sem1, out_sem0, out_sem1):
    xin_bufs = ((xin00, xin01, xin02), (xin10, xin11, xin12))
    yout_bufs = ((yo00, yo01, yo02), (yo10, yo11, yo12))
    cid = lax.axis_index("c")
    sid = lax.axis_index("s")
    wid = sid * _NC + cid
    row0 = wid * _ROWS_PER_W

    in_sems = (in_sem0, in_sem1)
    out_sems = (out_sem0, out_sem1)

    def start_in(off_nrows, slot):
        off, nrows = off_nrows
        return [
            pltpu.async_copy(x_hbm.at[row0 + off + r], xin_bufs[slot][r],
                             in_sems[slot])
            for r in range(nrows)
        ]

    def start_out(off_nrows, slot):
        off, nrows = off_nrows
        return [
            pltpu.async_copy(yout_bufs[slot][r], out_hbm.at[row0 + off + r],
                             out_sems[slot])
            for r in range(nrows)
        ]

    d_in = {0: start_in(_STEPS[0], 0), 1: start_in(_STEPS[1], 1)}

    nbase = sid * _SLICE
    w_v = yo01
    pltpu.sync_copy(w_hbm.at[pl.ds(nbase * 16, _SLICE * 16)], w_v)
    pltpu.sync_copy(ia_hbm.at[pl.ds(nbase, _SLICE)], iasl_v)
    pltpu.sync_copy(ib_hbm.at[pl.ds(nbase, _SLICE)], ibsl_v)

    lane16 = lax.iota(jnp.int32, _L) * 16

    @plsc.parallel_loop(0, _NGRP, 1, unroll=2)
    def _grp(g):
        goff = g * _L
        wgt = [plsc.load_gather(w_v, [lane16 + (goff * 16 + t)])
               for t in range(16)]
        if _TAU != 1.0:
            wgt = [w * (1.0 / _TAU) for w in wgt]
        m = wgt[0]
        for t in range(1, 16):
            m = jnp.maximum(m, wgt[t])
        e = [jnp.exp(w - m) for w in wgt]
        tot = e[0]
        for t in range(1, 16):
            tot = tot + e[t]
        rinv = 1.0 / tot
        p = [v * rinv for v in e]
        s23 = p[2] + p[3]
        s45 = p[4] + p[5]
        s67 = p[6] + p[7]
        s89 = p[8] + p[9]
        s1011 = p[10] + p[11]
        s1213 = p[12] + p[13]
        s1415 = p[14] + p[15]
        c0 = (s89 + s1011) + (s1213 + s1415)
        ca = (s23 + s67) - (s89 + s1213)
        cb = (s45 + s67) - (s89 + s1011)
        cab = (((p[1] - p[2]) - (p[4] + p[7])) - 2.0 * p[6]
               + (p[8] + 2.0 * p[9]) + (p[11] + p[13]) - p[14])
        u0 = plsc.bitcast(c0, jnp.int32)
        ua = plsc.bitcast(ca, jnp.int32)
        ub = plsc.bitcast(cb, jnp.int32)
        uab = plsc.bitcast(cab, jnp.int32)
        pc1sl_v[pl.ds(goff, _L)] = _round_bf16_lo(u0) | _round_bf16_hi(ua)
        pc2sl_v[pl.ds(goff, _L)] = _round_bf16_lo(ub) | _round_bf16_hi(uab)
        pksl_v[pl.ds(goff, _L)] = (iasl_v[pl.ds(goff, _L)]
                                   | jnp.left_shift(ibsl_v[pl.ds(goff, _L)],
                                                    16))

    pltpu.sync_copy(pksl_v, pk_sh.at[pl.ds(nbase, _SLICE)])
    pltpu.sync_copy(pc1sl_v, pc1_sh.at[pl.ds(nbase, _SLICE)])
    pltpu.sync_copy(pc2sl_v, pc2_sh.at[pl.ds(nbase, _SLICE)])
    plsc.subcore_barrier()
    pltpu.sync_copy(pk_sh, pk_v)
    pltpu.sync_copy(pc1_sh, pc1_v)
    pltpu.sync_copy(pc2_sh, pc2_v)

    def compute(slot, nrows):
        @plsc.parallel_loop(0, _NCHUNK, 1, unroll=4)
        def _chunk(jc):
            off = jc * _L
            pab = pk_v[pl.ds(off, _L)]
            ia = jnp.bitwise_and(pab, 0xFFFF)
            ib = lax.shift_right_logical(pab, 16)
            p1 = pc1_v[pl.ds(off, _L)]
            p2 = pc2_v[pl.ds(off, _L)]
            c0 = plsc.bitcast(jnp.left_shift(p1, 16), jnp.float32)
            ca = plsc.bitcast(p1, jnp.float32)
            cb = plsc.bitcast(jnp.left_shift(p2, 16), jnp.float32)
            cab = plsc.bitcast(p2, jnp.float32)
            for r in range(nrows):
                a = plsc.load_gather(xin_bufs[slot][r], [ia])
                b = plsc.load_gather(xin_bufs[slot][r], [ib])
                yout_bufs[slot][r][pl.ds(off, _L)] = (
                    (c0 + cb * b) + a * (ca + cab * b))

    nsteps = len(_STEPS)
    d_out = {}
    for g in range(nsteps):
        slot = g % 2
        if g >= 2:
            for d in d_out[slot]:
                d.wait()
        for d in d_in[slot]:
            d.wait()
        compute(slot, _STEPS[g][1])
        d_out[slot] = start_out(_STEPS[g], slot)
        if g + 2 < nsteps:
            d_in[slot] = start_in(_STEPS[g + 2], slot)
    for s in (0, 1):
        for d in d_out[s]:
            d.wait()


@functools.partial(jax.jit, donate_argnums=())
def _logic_sc(x, weight, idx_a, idx_b):
    mesh = plsc.VectorSubcoreMesh(
        core_axis_name="c", subcore_axis_name="s",
        num_cores=_NC, num_subcores=_NS)
    fn = pl.kernel(
        _sc_body,
        out_type=jax.ShapeDtypeStruct((_BATCH, _OUT_DIM), jnp.float32),
        mesh=mesh,
        compiler_params=pltpu.CompilerParams(needs_layout_passes=False),
        scratch_types=[
            pltpu.VMEM((_OUT_DIM,), jnp.int32),
            pltpu.VMEM((_OUT_DIM,), jnp.int32),
            pltpu.VMEM((_OUT_DIM,), jnp.int32),
            pltpu.VMEM((_SLICE,), jnp.int32),
            pltpu.VMEM((_SLICE,), jnp.int32),
            pltpu.VMEM((_SLICE,), jnp.int32),
            pltpu.VMEM((_SLICE,), jnp.int32),
            pltpu.VMEM((_SLICE,), jnp.int32),
            pltpu.VMEM_SHARED((_OUT_DIM,), jnp.int32),
            pltpu.VMEM_SHARED((_OUT_DIM,), jnp.int32),
            pltpu.VMEM_SHARED((_OUT_DIM,), jnp.int32),
            pltpu.VMEM((_IN_DIM,), jnp.float32),
            pltpu.VMEM((_IN_DIM,), jnp.float32),
            pltpu.VMEM((_IN_DIM,), jnp.float32),
            pltpu.VMEM((_IN_DIM,), jnp.float32),
            pltpu.VMEM((_IN_DIM,), jnp.float32),
            pltpu.VMEM((_IN_DIM,), jnp.float32),
            pltpu.VMEM((_OUT_DIM,), jnp.float32),
            pltpu.VMEM((_OUT_DIM,), jnp.float32),
            pltpu.VMEM((_OUT_DIM,), jnp.float32),
            pltpu.VMEM((_OUT_DIM,), jnp.float32),
            pltpu.VMEM((_OUT_DIM,), jnp.float32),
            pltpu.VMEM((_OUT_DIM,), jnp.float32),
            pltpu.SemaphoreType.DMA,
            pltpu.SemaphoreType.DMA,
            pltpu.SemaphoreType.DMA,
            pltpu.SemaphoreType.DMA,
        ],
    )
    return fn(x, weight.reshape(_OUT_DIM * 16),
              idx_a.astype(jnp.int32), idx_b.astype(jnp.int32))


def kernel(x, weight, idx_a, idx_b):
    return _logic_sc(x, weight, idx_a, idx_b)


# speedup vs baseline: 1.0000x; 1.0000x over previous
import functools

import jax
import jax.numpy as jnp
from jax import lax
from jax.experimental import pallas as pl
from jax.experimental.pallas import tpu as pltpu
from jax.experimental.pallas import tpu_sc as plsc

_BATCH = 2048
_IN_DIM = 8192
_OUT_DIM = 8192
_TAU = 1.0

_NC = 2
_NS = 16
_NW = _NC * _NS
_L = 16

_ROWS_PER_W = _BATCH // _NW
_R = 3
_NCHUNK = _OUT_DIM // _L
_STEPS = [(3 * k, 3) for k in range(21)] + [(63, 1)]

_SLICE = _OUT_DIM // _NS
_NGRP = _SLICE // _L


def _round_bf16_lo(u):
    return lax.shift_right_logical(u + jnp.int32(0x8000), 16)


def _round_bf16_hi(u):
    return jnp.bitwise_and(u + jnp.int32(0x8000), jnp.int32(-65536))


def _sc_body(x_hbm, w_hbm, ia_hbm, ib_hbm, out_hbm,
             pk_v, pc1_v, pc2_v,
             iasl_v, ibsl_v, pksl_v, pc1sl_v, pc2sl_v,
             pk_sh, pc1_sh, pc2_sh,
             xin00, xin01, xin02, xin10, xin11, xin12,
             yo00, yo01, yo02, yo10, yo11, yo12,
             in_sem0, in_sem1, out_sem0, out_sem1):
    xin_bufs = ((xin00, xin01, xin02), (xin10, xin11, xin12))
    yout_bufs = ((yo00, yo01, yo02), (yo10, yo11, yo12))
    cid = lax.axis_index("c")
    sid = lax.axis_index("s")
    wid = sid * _NC + cid
    row0 = wid * _ROWS_PER_W

    in_sems = (in_sem0, in_sem1)
    out_sems = (out_sem0, out_sem1)

    def start_in(off_nrows, slot):
        off, nrows = off_nrows
        return [
            pltpu.async_copy(x_hbm.at[row0 + off + r], xin_bufs[slot][r],
                             in_sems[slot])
            for r in range(nrows)
        ]

    def start_out(off_nrows, slot):
        off, nrows = off_nrows
        return [
            pltpu.async_copy(yout_bufs[slot][r], out_hbm.at[row0 + off + r],
                             out_sems[slot])
            for r in range(nrows)
        ]

    d_in = {0: start_in(_STEPS[0], 0), 1: start_in(_STEPS[1], 1)}

    nbase = sid * _SLICE
    w_v = yo01
    pltpu.sync_copy(w_hbm.at[pl.ds(nbase * 16, _SLICE * 16)], w_v)
    pltpu.sync_copy(ia_hbm.at[pl.ds(nbase, _SLICE)], iasl_v)
    pltpu.sync_copy(ib_hbm.at[pl.ds(nbase, _SLICE)], ibsl_v)

    lane16 = lax.iota(jnp.int32, _L) * 16

    @plsc.parallel_loop(0, _NGRP, 1, unroll=2)
    def _grp(g):
        goff = g * _L
        wgt = [plsc.load_gather(w_v, [lane16 + (goff * 16 + t)])
               for t in range(16)]
        if _TAU != 1.0:
            wgt = [w * (1.0 / _TAU) for w in wgt]
        m = wgt[0]
        for t in range(1, 16):
            m = jnp.maximum(m, wgt[t])
        e = [jnp.exp(w - m) for w in wgt]
        tot = e[0]
        for t in range(1, 16):
            tot = tot + e[t]
        rinv = 1.0 / tot
        p = [v * rinv for v in e]
        s23 = p[2] + p[3]
        s45 = p[4] + p[5]
        s67 = p[6] + p[7]
        s89 = p[8] + p[9]
        s1011 = p[10] + p[11]
        s1213 = p[12] + p[13]
        s1415 = p[14] + p[15]
        c0 = (s89 + s1011) + (s1213 + s1415)
        ca = (s23 + s67) - (s89 + s1213)
        cb = (s45 + s67) - (s89 + s1011)
        cab = (((p[1] - p[2]) - (p[4] + p[7])) - 2.0 * p[6]
               + (p[8] + 2.0 * p[9]) + (p[11] + p[13]) - p[14])
        u0 = plsc.bitcast(c0, jnp.int32)
        ua = plsc.bitcast(ca, jnp.int32)
        ub = plsc.bitcast(cb, jnp.int32)
        uab = plsc.bitcast(cab, jnp.int32)
        pc1sl_v[pl.ds(goff, _L)] = _round_bf16_lo(u0) | _round_bf16_hi(ua)
        pc2sl_v[pl.ds(goff, _L)] = _round_bf16_lo(ub) | _round_bf16_hi(uab)
        pksl_v[pl.ds(goff, _L)] = (iasl_v[pl.ds(goff, _L)]
                                   | jnp.left_shift(ibsl_v[pl.ds(goff, _L)],
                                                    16))

    pltpu.sync_copy(pksl_v, pk_sh.at[pl.ds(nbase, _SLICE)])
    pltpu.sync_copy(pc1sl_v, pc1_sh.at[pl.ds(nbase, _SLICE)])
    pltpu.sync_copy(pc2sl_v, pc2_sh.at[pl.ds(nbase, _SLICE)])
    plsc.subcore_barrier()
    pltpu.sync_copy(pk_sh, pk_v)
    pltpu.sync_copy(pc1_sh, pc1_v)
    pltpu.sync_copy(pc2_sh, pc2_v)

    def compute(slot, nrows):
        @plsc.parallel_loop(0, _NCHUNK, 1, unroll=4)
        def _chunk(jc):
            off = jc * _L
            pab = pk_v[pl.ds(off, _L)]
            ia = jnp.bitwise_and(pab, 0xFFFF)
            ib = lax.shift_right_logical(pab, 16)
            p1 = pc1_v[pl.ds(off, _L)]
            p2 = pc2_v[pl.ds(off, _L)]
            c0 = plsc.bitcast(jnp.left_shift(p1, 16), jnp.float32)
            ca = plsc.bitcast(p1, jnp.float32)
            cb = plsc.bitcast(jnp.left_shift(p2, 16), jnp.float32)
            cab = plsc.bitcast(p2, jnp.float32)
            for r in range(nrows):
                a = plsc.load_gather(xin_bufs[slot][r], [ia])
                b = plsc.load_gather(xin_bufs[slot][r], [ib])
                yout_bufs[slot][r][pl.ds(off, _L)] = (
                    (c0 + cb * b) + a * (ca + cab * b))

    nsteps = len(_STEPS)
    d_out = {}
    for g in range(nsteps):
        slot = g % 2
        if g >= 2:
            for d in d_out[slot]:
                d.wait()
        for d in d_in[slot]:
            d.wait()
        compute(slot, _STEPS[g][1])
        d_out[slot] = start_out(_STEPS[g], slot)
        if g + 2 < nsteps:
            d_in[slot] = start_in(_STEPS[g + 2], slot)
    for s in (0, 1):
        for d in d_out[s]:
            d.wait()


@functools.partial(jax.jit, donate_argnums=())
def _logic_sc(x, weight, idx_a, idx_b):
    mesh = plsc.VectorSubcoreMesh(
        core_axis_name="c", subcore_axis_name="s",
        num_cores=_NC, num_subcores=_NS)
    fn = pl.kernel(
        _sc_body,
        out_type=jax.ShapeDtypeStruct((_BATCH, _OUT_DIM), jnp.float32),
        mesh=mesh,
        compiler_params=pltpu.CompilerParams(
            needs_layout_passes=False, skip_device_barrier=True),
        scratch_types=[
            pltpu.VMEM((_OUT_DIM,), jnp.int32),
            pltpu.VMEM((_OUT_DIM,), jnp.int32),
            pltpu.VMEM((_OUT_DIM,), jnp.int32),
            pltpu.VMEM((_SLICE,), jnp.int32),
            pltpu.VMEM((_SLICE,), jnp.int32),
            pltpu.VMEM((_SLICE,), jnp.int32),
            pltpu.VMEM((_SLICE,), jnp.int32),
            pltpu.VMEM((_SLICE,), jnp.int32),
            pltpu.VMEM_SHARED((_OUT_DIM,), jnp.int32),
            pltpu.VMEM_SHARED((_OUT_DIM,), jnp.int32),
            pltpu.VMEM_SHARED((_OUT_DIM,), jnp.int32),
            pltpu.VMEM((_IN_DIM,), jnp.float32),
            pltpu.VMEM((_IN_DIM,), jnp.float32),
            pltpu.VMEM((_IN_DIM,), jnp.float32),
            pltpu.VMEM((_IN_DIM,), jnp.float32),
            pltpu.VMEM((_IN_DIM,), jnp.float32),
            pltpu.VMEM((_IN_DIM,), jnp.float32),
            pltpu.VMEM((_OUT_DIM,), jnp.float32),
            pltpu.VMEM((_OUT_DIM,), jnp.float32),
            pltpu.VMEM((_OUT_DIM,), jnp.float32),
            pltpu.VMEM((_OUT_DIM,), jnp.float32),
            pltpu.VMEM((_OUT_DIM,), jnp.float32),
            pltpu.VMEM((_OUT_DIM,), jnp.float32),
            pltpu.SemaphoreType.DMA,
            pltpu.SemaphoreType.DMA,
            pltpu.SemaphoreType.DMA,
            pltpu.SemaphoreType.DMA,
        ],
    )
    return fn(x, weight.reshape(_OUT_DIM * 16),
              idx_a.astype(jnp.int32), idx_b.astype(jnp.int32))


def kernel(x, weight, idx_a, idx_b):
    return _logic_sc(x, weight, idx_a, idx_b)


# speedup vs baseline: 1.0527x; 1.0527x over previous
import functools

import jax
import jax.numpy as jnp
from jax import lax
from jax.experimental import pallas as pl
from jax.experimental.pallas import tpu as pltpu
from jax.experimental.pallas import tpu_sc as plsc

_BATCH = 2048
_IN_DIM = 8192
_OUT_DIM = 8192
_TAU = 1.0

_NC = 2
_NS = 16
_NW = _NC * _NS
_L = 16

_ROWS_PER_W = _BATCH // _NW
_R = 3
_NCHUNK = _OUT_DIM // _L
_STEPS = [(3 * k, 3) for k in range(21)] + [(63, 1)]

_SLICE = _OUT_DIM // _NS
_NGRP = _SLICE // _L


def _round_bf16_lo(u):
    return lax.shift_right_logical(u + jnp.int32(0x8000), 16)


def _round_bf16_hi(u):
    return jnp.bitwise_and(u + jnp.int32(0x8000), jnp.int32(-65536))


def _sc_body(x_hbm, w_hbm, ia_hbm, ib_hbm, out_hbm,
             pk_v, pc1_v, pc2_v,
             iasl_v, ibsl_v, pksl_v, pc1sl_v, pc2sl_v,
             pk_sh, pc1_sh, pc2_sh,
             xin00, xin01, xin02, xin10, xin11, xin12,
             yo00, yo01, yo02, yo10, yo11, yo12,
             in_sem0, in_sem1, out_sem0, out_sem1):
    xin_bufs = ((xin00, xin01, xin02), (xin10, xin11, xin12))
    yout_bufs = ((yo00, yo01, yo02), (yo10, yo11, yo12))
    cid = lax.axis_index("c")
    sid = lax.axis_index("s")
    wid = sid * _NC + cid
    row0 = wid * _ROWS_PER_W

    in_sems = (in_sem0, in_sem1)
    out_sems = (out_sem0, out_sem1)

    last = row0 + _ROWS_PER_W - 1

    def start_in(off, slot, nrows=_R):
        return [
            pltpu.async_copy(
                x_hbm.at[jnp.minimum(row0 + off + r, last)],
                xin_bufs[slot][r], in_sems[slot])
            for r in range(nrows)
        ]

    def wait_in(slot, nrows=_R):
        for r in range(nrows):
            pltpu.make_async_copy(x_hbm.at[row0], xin_bufs[slot][r],
                                  in_sems[slot]).wait()

    def start_out(off, slot, nrows=_R):
        return [
            pltpu.async_copy(yout_bufs[slot][r], out_hbm.at[row0 + off + r],
                             out_sems[slot])
            for r in range(nrows)
        ]

    def wait_out(slot, nrows=_R):
        for r in range(nrows):
            pltpu.make_async_copy(yout_bufs[slot][r], out_hbm.at[row0],
                                  out_sems[slot]).wait()

    start_in(0, 0)
    start_in(3, 1)

    nbase = sid * _SLICE
    w_v = yo01
    pltpu.sync_copy(w_hbm.at[pl.ds(nbase * 16, _SLICE * 16)], w_v)
    pltpu.sync_copy(ia_hbm.at[pl.ds(nbase, _SLICE)], iasl_v)
    pltpu.sync_copy(ib_hbm.at[pl.ds(nbase, _SLICE)], ibsl_v)

    lane16 = lax.iota(jnp.int32, _L) * 16

    @plsc.parallel_loop(0, _NGRP, 1, unroll=2)
    def _grp(g):
        goff = g * _L
        wgt = [plsc.load_gather(w_v, [lane16 + (goff * 16 + t)])
               for t in range(16)]
        if _TAU != 1.0:
            wgt = [w * (1.0 / _TAU) for w in wgt]
        m = wgt[0]
        for t in range(1, 16):
            m = jnp.maximum(m, wgt[t])
        e = [jnp.exp(w - m) for w in wgt]
        tot = e[0]
        for t in range(1, 16):
            tot = tot + e[t]
        rinv = 1.0 / tot
        p = [v * rinv for v in e]
        s23 = p[2] + p[3]
        s45 = p[4] + p[5]
        s67 = p[6] + p[7]
        s89 = p[8] + p[9]
        s1011 = p[10] + p[11]
        s1213 = p[12] + p[13]
        s1415 = p[14] + p[15]
        c0 = (s89 + s1011) + (s1213 + s1415)
        ca = (s23 + s67) - (s89 + s1213)
        cb = (s45 + s67) - (s89 + s1011)
        cab = (((p[1] - p[2]) - (p[4] + p[7])) - 2.0 * p[6]
               + (p[8] + 2.0 * p[9]) + (p[11] + p[13]) - p[14])
        u0 = plsc.bitcast(c0, jnp.int32)
        ua = plsc.bitcast(ca, jnp.int32)
        ub = plsc.bitcast(cb, jnp.int32)
        uab = plsc.bitcast(cab, jnp.int32)
        pc1sl_v[pl.ds(goff, _L)] = _round_bf16_lo(u0) | _round_bf16_hi(ua)
        pc2sl_v[pl.ds(goff, _L)] = _round_bf16_lo(ub) | _round_bf16_hi(uab)
        pksl_v[pl.ds(goff, _L)] = (iasl_v[pl.ds(goff, _L)]
                                   | jnp.left_shift(ibsl_v[pl.ds(goff, _L)],
                                                    16))

    pltpu.sync_copy(pksl_v, pk_sh.at[pl.ds(nbase, _SLICE)])
    pltpu.sync_copy(pc1sl_v, pc1_sh.at[pl.ds(nbase, _SLICE)])
    pltpu.sync_copy(pc2sl_v, pc2_sh.at[pl.ds(nbase, _SLICE)])
    plsc.subcore_barrier()
    pltpu.sync_copy(pk_sh, pk_v)
    pltpu.sync_copy(pc1_sh, pc1_v)
    pltpu.sync_copy(pc2_sh, pc2_v)

    def compute(slot, nrows):
        @plsc.parallel_loop(0, _NCHUNK, 1, unroll=4)
        def _chunk(jc):
            off = jc * _L
            pab = pk_v[pl.ds(off, _L)]
            ia = jnp.bitwise_and(pab, 0xFFFF)
            ib = lax.shift_right_logical(pab, 16)
            p1 = pc1_v[pl.ds(off, _L)]
            p2 = pc2_v[pl.ds(off, _L)]
            c0 = plsc.bitcast(jnp.left_shift(p1, 16), jnp.float32)
            ca = plsc.bitcast(p1, jnp.float32)
            cb = plsc.bitcast(jnp.left_shift(p2, 16), jnp.float32)
            cab = plsc.bitcast(p2, jnp.float32)
            for r in range(nrows):
                a = plsc.load_gather(xin_bufs[slot][r], [ia])
                b = plsc.load_gather(xin_bufs[slot][r], [ib])
                yout_bufs[slot][r][pl.ds(off, _L)] = (
                    (c0 + cb * b) + a * (ca + cab * b))

    wait_in(0)
    compute(0, _R)
    start_out(0, 0)
    start_in(6, 0)
    wait_in(1)
    compute(1, _R)
    start_out(3, 1)
    start_in(9, 1)

    def pair_body(p, carry):
        off = p * 6
        wait_out(0)
        wait_in(0)
        compute(0, _R)
        start_out(off, 0)
        start_in(off + 6, 0)
        wait_out(1)
        wait_in(1)
        compute(1, _R)
        start_out(off + 3, 1)
        start_in(off + 9, 1)
        return carry

    lax.fori_loop(1, 10, pair_body, 0)

    wait_out(0)
    wait_in(0)
    compute(0, _R)
    start_out(60, 0)
    wait_out(1)
    wait_in(1)
    compute(1, 1)
    start_out(63, 1, nrows=1)
    wait_out(0)
    wait_out(1, nrows=1)


@functools.partial(jax.jit, donate_argnums=())
def _logic_sc(x, weight, idx_a, idx_b):
    mesh = plsc.VectorSubcoreMesh(
        core_axis_name="c", subcore_axis_name="s",
        num_cores=_NC, num_subcores=_NS)
    fn = pl.kernel(
        _sc_body,
        out_type=jax.ShapeDtypeStruct((_BATCH, _OUT_DIM), jnp.float32),
        mesh=mesh,
        compiler_params=pltpu.CompilerParams(needs_layout_passes=False),
        scratch_types=[
            pltpu.VMEM((_OUT_DIM,), jnp.int32),
            pltpu.VMEM((_OUT_DIM,), jnp.int32),
            pltpu.VMEM((_OUT_DIM,), jnp.int32),
            pltpu.VMEM((_SLICE,), jnp.int32),
            pltpu.VMEM((_SLICE,), jnp.int32),
            pltpu.VMEM((_SLICE,), jnp.int32),
            pltpu.VMEM((_SLICE,), jnp.int32),
            pltpu.VMEM((_SLICE,), jnp.int32),
            pltpu.VMEM_SHARED((_OUT_DIM,), jnp.int32),
            pltpu.VMEM_SHARED((_OUT_DIM,), jnp.int32),
            pltpu.VMEM_SHARED((_OUT_DIM,), jnp.int32),
            pltpu.VMEM((_IN_DIM,), jnp.float32),
            pltpu.VMEM((_IN_DIM,), jnp.float32),
            pltpu.VMEM((_IN_DIM,), jnp.float32),
            pltpu.VMEM((_IN_DIM,), jnp.float32),
            pltpu.VMEM((_IN_DIM,), jnp.float32),
            pltpu.VMEM((_IN_DIM,), jnp.float32),
            pltpu.VMEM((_OUT_DIM,), jnp.float32),
            pltpu.VMEM((_OUT_DIM,), jnp.float32),
            pltpu.VMEM((_OUT_DIM,), jnp.float32),
            pltpu.VMEM((_OUT_DIM,), jnp.float32),
            pltpu.VMEM((_OUT_DIM,), jnp.float32),
            pltpu.VMEM((_OUT_DIM,), jnp.float32),
            pltpu.SemaphoreType.DMA,
            pltpu.SemaphoreType.DMA,
            pltpu.SemaphoreType.DMA,
            pltpu.SemaphoreType.DMA,
        ],
    )
    return fn(x, weight.reshape(_OUT_DIM * 16),
              idx_a.astype(jnp.int32), idx_b.astype(jnp.int32))


def kernel(x, weight, idx_a, idx_b):
    return _logic_sc(x, weight, idx_a, idx_b)
